# Initial kernel scaffold; baseline (speedup 1.0000x reference)
import jax
import jax.numpy as jnp
from jax.experimental import pallas as pl


def kernel(input_ids, wte, wpe):
    raise NotImplementedError("write your pallas kernel here")


# speedup vs baseline: 1.0351x; 1.0351x over previous
import functools

import jax
import jax.numpy as jnp
from jax import lax
from jax.experimental import pallas as pl
from jax.experimental.pallas import tpu as pltpu
from jax.experimental.pallas import tpu_sc as plsc

VOCAB = 50257
HIDDEN = 768
MAX_POS = 2048
BATCH = 4
SEQ = 2048
TOK = BATCH * SEQ
LANES = 16
NC, NS = 2, 16
NW = NC * NS
TPW = TOK // NW
CH = 64
NCH = TPW // CH
HV = HIDDEN // LANES

_mesh = plsc.VectorSubcoreMesh(core_axis_name="c", subcore_axis_name="s")


@functools.partial(
    pl.kernel,
    mesh=_mesh,
    out_type=jax.ShapeDtypeStruct((TOK, HIDDEN), jnp.float32),
    scratch_types=[
        pltpu.VMEM((NCH, CH), jnp.int32),
        pltpu.VMEM((CH, HIDDEN), jnp.float32),
        pltpu.VMEM((CH, HIDDEN), jnp.float32),
        pltpu.SemaphoreType.DMA,
        pltpu.SemaphoreType.DMA,
    ],
)
def _embed(ids_hbm, wte_hbm, wpe_hbm, out_hbm, idx_v, rows_v, wpe_v, gsem, psem):
    wid = lax.axis_index("s") * NC + lax.axis_index("c")
    base = wid * TPW
    pos_base = lax.rem(base, SEQ)
    pltpu.sync_copy(ids_hbm.at[wid], idx_v)
    for c in range(NCH):
        g = pltpu.async_copy(wte_hbm.at[idx_v.at[c]], rows_v, gsem)
        p = pltpu.async_copy(wpe_hbm.at[pl.ds(pos_base + c * CH, CH)], wpe_v, psem)
        g.wait()
        p.wait()

        def add_row(r, carry):
            for j in range(HV):
                s = pl.ds(j * LANES, LANES)
                rows_v[r, s] = rows_v[r, s] + wpe_v[r, s]
            return carry

        lax.fori_loop(0, CH, add_row, 0)
        pltpu.sync_copy(rows_v, out_hbm.at[pl.ds(base + c * CH, CH)])


def kernel(input_ids, wte, wpe):
    ids = input_ids.reshape(NW, NCH, CH).astype(jnp.int32)
    out = _embed(ids, wte, wpe)
    return out.reshape(BATCH, SEQ, HIDDEN)


# speedup vs baseline: 1.0882x; 1.0513x over previous
import functools

import jax
import jax.numpy as jnp
from jax import lax
from jax.experimental import pallas as pl
from jax.experimental.pallas import tpu as pltpu
from jax.experimental.pallas import tpu_sc as plsc

VOCAB = 50257
HIDDEN = 768
MAX_POS = 2048
BATCH = 4
SEQ = 2048
TOK = BATCH * SEQ
LANES = 16
NC, NS = 2, 16
NW = NC * NS
PPW = SEQ // NW
CH = 32
CPB = PPW // CH
NCH = BATCH * CPB
HV = HIDDEN // LANES
NBUF = 3

_mesh = plsc.VectorSubcoreMesh(core_axis_name="c", subcore_axis_name="s")


@functools.partial(
    pl.kernel,
    mesh=_mesh,
    out_type=jax.ShapeDtypeStruct((TOK, HIDDEN), jnp.float32),
    scratch_types=[
        pltpu.VMEM((BATCH, PPW), jnp.int32),
        pltpu.VMEM((PPW, HIDDEN), jnp.float32),
        pltpu.VMEM((CH, HIDDEN), jnp.float32),
        pltpu.VMEM((CH, HIDDEN), jnp.float32),
        pltpu.VMEM((CH, HIDDEN), jnp.float32),
        pltpu.SemaphoreType.DMA,
        pltpu.SemaphoreType.DMA,
        pltpu.SemaphoreType.DMA,
        pltpu.SemaphoreType.DMA,
        pltpu.SemaphoreType.DMA,
        pltpu.SemaphoreType.DMA,
        pltpu.SemaphoreType.DMA,
    ],
)
def _embed(ids_hbm, wte_hbm, wpe_hbm, out_hbm, idx_v, wpe_v, rb0, rb1, rb2,
           wsem, gs0, gs1, gs2, os0, os1, os2):
    wid = lax.axis_index("s") * NC + lax.axis_index("c")
    pos0 = wid * PPW
    rows = [rb0, rb1, rb2]
    gsems = [gs0, gs1, gs2]
    osems = [os0, os1, os2]

    wp = pltpu.async_copy(wpe_hbm.at[pl.ds(pos0, PPW)], wpe_v, wsem)
    for b in range(BATCH):
        pltpu.sync_copy(ids_hbm.at[b, wid], idx_v.at[b])

    def gather(k):
        b, h = divmod(k, CPB)
        return pltpu.async_copy(
            wte_hbm.at[idx_v.at[b, pl.ds(h * CH, CH)]], rows[k % NBUF],
            gsems[k % NBUF])

    g = {k: gather(k) for k in range(min(2, NCH))}
    o = {}
    wp.wait()
    for k in range(NCH):
        b, h = divmod(k, CPB)
        buf = k % NBUF
        g[k].wait()

        def add_row(r, carry):
            for j in range(HV):
                s = pl.ds(j * LANES, LANES)
                rows[buf][r, s] = rows[buf][r, s] + wpe_v[h * CH + r, s]
            return carry

        lax.fori_loop(0, CH, add_row, 0)
        o[k] = pltpu.async_copy(
            rows[buf], out_hbm.at[pl.ds(b * SEQ + pos0 + h * CH, CH)],
            osems[buf])
        if k + 2 < NCH:
            if k - 1 >= 0:
                o[k - 1].wait()
            g[k + 2] = gather(k + 2)
    o[NCH - 2].wait()
    o[NCH - 1].wait()


def kernel(input_ids, wte, wpe):
    ids = input_ids.reshape(BATCH, NW, PPW).astype(jnp.int32)
    out = _embed(ids, wte, wpe)
    return out.reshape(BATCH, SEQ, HIDDEN)
